# baseline (speedup 1.0000x reference)
import jax
import jax.numpy as jnp
from jax import lax
from jax.experimental import pallas as pl
from jax.experimental.pallas import tpu as pltpu
from jax.experimental.pallas import tpu_sc as plsc

N = 10000
E = 320000
D = 128
R = 64
B = 2

NPAD = 10240
NTILE = 16
EPT = E // NTILE
ECHUNK = 4000
NCHUNK = EPT // ECHUNK
CVECS = ECHUNK // 16
UNROLL = 25
NODES_PER_TILE = NPAD // NTILE

LN2 = 0.6931471805599453
MANT_MASK = 0x007FFFFF
ONE_BITS = 0x3F800000


def _prologue_body(ri_ref, rel_ref, w_ref, aux_ref):
    ri = ri_ref[...]
    rel = rel_ref[...]
    w = w_ref[...]
    iota_r = lax.broadcasted_iota(jnp.int32, (8, R), 1)
    oh = (ri[:, :R] == iota_r).astype(jnp.float32)
    query = jnp.dot(oh, rel, preferred_element_type=jnp.float32)
    wq = lax.dot_general(query, w, (((1,), (1,)), ((), ())),
                         preferred_element_type=jnp.float32)
    coef = lax.dot_general(query * wq, rel, (((1,), (1,)), ((), ())),
                           preferred_element_type=jnp.float32)
    q2 = jnp.sum(query * query, axis=1, keepdims=True)
    aux_ref[...] = jnp.concatenate(
        [coef, jnp.broadcast_to(q2, (8, 16)),
         jnp.zeros((8, D - R - 16), jnp.float32)], axis=1)


def _sc_body(h_hbm, src_hbm, dst_hbm, et_hbm, aux_hbm, rb_hbm,
             out_hbm,
             hv, cv, rvv,
             srcva, dstva, etva, srcvb, dstvb, etvb,
             sumacc, maxacc, done_s, bsum, bmax, outv,
             shsum, shmax, sema, semb):
    b = lax.axis_index("c")
    s = lax.axis_index("s")
    ebase = s * EPT

    bufsets = ((srcva, dstva, etva), (srcvb, dstvb, etvb))
    sems = (sema, semb)

    def start_chunk(ci):
        cbase = ebase + ci * ECHUNK
        bs = bufsets[ci % 2]
        sem = sems[ci % 2]
        return [pltpu.async_copy(src_hbm.at[pl.ds(cbase, ECHUNK)], bs[0], sem),
                pltpu.async_copy(dst_hbm.at[pl.ds(cbase, ECHUNK)], bs[1], sem),
                pltpu.async_copy(et_hbm.at[pl.ds(cbase, ECHUNK)], bs[2], sem)]

    handles = {0: start_chunk(0), 1: start_chunk(1)}

    pltpu.sync_copy(rb_hbm.at[b], rvv)
    pltpu.sync_copy(h_hbm.at[b], hv)
    pltpu.sync_copy(aux_hbm.at[b, pl.ds(0, R + 16)], cv)
    rv = rvv[...]

    zeros16 = jnp.zeros((16,), jnp.float32)
    neg16 = jnp.full((16,), -1e9, jnp.float32)

    def init_body(i, c):
        base = i * 128
        for u in range(8):
            sumacc[pl.ds(base + u * 16, 16)] = zeros16
            maxacc[pl.ds(base + u * 16, 16)] = neg16
        return c

    lax.fori_loop(0, NPAD // 128, init_body, 0)

    def edge_vec(sv, dv, tv, off):
        s16 = sv[pl.ds(off, 16)]
        d16 = dv[pl.ds(off, 16)]
        t16 = tv[pl.ds(off, 16)]
        hb = plsc.load_gather(hv, [s16])
        cf = plsc.load_gather(cv, [t16])
        plsc.addupdate_scatter(sumacc, [d16], hb * cf)
        m = t16 == rv
        cur = plsc.load_gather(maxacc, [d16])
        plsc.store_scatter(maxacc, [d16], jnp.maximum(cur, hb), mask=m)
        chk = plsc.load_gather(maxacc, [d16])
        return m & (chk < hb)

    def fix_vec(sv, dv, tv, off):
        s16 = sv[pl.ds(off, 16)]
        d16 = dv[pl.ds(off, 16)]
        t16 = tv[pl.ds(off, 16)]
        hb = plsc.load_gather(hv, [s16])
        m = t16 == rv
        chk = plsc.load_gather(maxacc, [d16])
        pend = m & (chk < hb)
        done_s[...] = jnp.where(pend, 0, 1)

        def retry(k, c2):
            p = done_s[...] == 0

            @pl.when(jnp.any(p))
            def _():
                cur2 = plsc.load_gather(maxacc, [d16])
                plsc.store_scatter(maxacc, [d16],
                                   jnp.maximum(cur2, hb), mask=p)
                chk2 = plsc.load_gather(maxacc, [d16])
                done_s[...] = jnp.where(p & (chk2 < hb), 0, 1)

            return c2

        lax.fori_loop(0, 15, retry, 0)

    def make_group_body(bufset):
        sv, dv, tv = bufset

        def group_body(i, c):
            base = i * (16 * UNROLL)
            pends = [edge_vec(sv, dv, tv, base + u * 16)
                     for u in range(UNROLL)]
            anyp = pends[0]
            for p in pends[1:]:
                anyp = anyp | p

            @pl.when(jnp.any(anyp))
            def _():
                for u in range(UNROLL):
                    fix_vec(sv, dv, tv, base + u * 16)

            return c

        return group_body

    for ci in range(NCHUNK):
        for hdl in handles.pop(ci):
            hdl.wait()
        lax.fori_loop(0, CVECS // UNROLL, make_group_body(bufsets[ci % 2]), 0)
        if ci + 2 < NCHUNK:
            handles[ci + 2] = start_chunk(ci + 2)

    pltpu.sync_copy(sumacc, shsum.at[s])
    pltpu.sync_copy(maxacc, shmax.at[s])
    plsc.subcore_barrier()

    nbase = s * NODES_PER_TILE
    pltpu.sync_copy(shsum.at[:, pl.ds(nbase, NODES_PER_TILE)], bsum)
    pltpu.sync_copy(shmax.at[:, pl.ds(nbase, NODES_PER_TILE)], bmax)

    q2 = cv[pl.ds(R, 16)]

    def comb_body(k, c):
        off = k * 16
        a = bsum[0, pl.ds(off, 16)]
        mx = bmax[0, pl.ds(off, 16)]
        for j in range(1, NTILE):
            a = a + bsum[j, pl.ds(off, 16)]
            mx = jnp.maximum(mx, bmax[j, pl.ds(off, 16)])

        hb16 = hv[pl.ds(nbase + off, 16)]
        score = a + hb16 * q2
        neural = 1.0 / (1.0 + jnp.exp(-score))
        sym = jnp.maximum(mx, 0.0)
        t = neural * sym
        u = (t + 1e-10) / (1.0 - t + 1e-10)
        bits = plsc.bitcast(u, jnp.int32)
        e = jnp.right_shift(bits, 23) - 127
        mant = plsc.bitcast(jnp.bitwise_or(jnp.bitwise_and(bits, MANT_MASK),
                                           ONE_BITS), jnp.float32)
        y = e.astype(jnp.float32) * LN2 + (mant - 1.0) * LN2
        y = y + u * jnp.exp(-y) - 1.0
        y = y + u * jnp.exp(-y) - 1.0
        y = y + u * jnp.exp(-y) - 1.0
        outv[pl.ds(off, 16)] = y
        return c

    lax.fori_loop(0, NODES_PER_TILE // 16, comb_body, 0)
    pltpu.sync_copy(outv, out_hbm.at[b, pl.ds(nbase, NODES_PER_TILE)])


def _build_sc_call():
    mesh = plsc.VectorSubcoreMesh(core_axis_name="c", subcore_axis_name="s")
    return pl.kernel(
        _sc_body,
        out_type=jax.ShapeDtypeStruct((B, NPAD), jnp.float32),
        mesh=mesh,
        compiler_params=pltpu.CompilerParams(needs_layout_passes=False),
        scratch_types=[
            pltpu.VMEM((NPAD,), jnp.float32),
            pltpu.VMEM((R + 16,), jnp.float32),
            pltpu.VMEM((16,), jnp.int32),
            pltpu.VMEM((ECHUNK,), jnp.int32),
            pltpu.VMEM((ECHUNK,), jnp.int32),
            pltpu.VMEM((ECHUNK,), jnp.int32),
            pltpu.VMEM((ECHUNK,), jnp.int32),
            pltpu.VMEM((ECHUNK,), jnp.int32),
            pltpu.VMEM((ECHUNK,), jnp.int32),
            pltpu.VMEM((NPAD,), jnp.float32),
            pltpu.VMEM((NPAD,), jnp.float32),
            pltpu.VMEM((16,), jnp.int32),
            pltpu.VMEM((NTILE, NODES_PER_TILE), jnp.float32),
            pltpu.VMEM((NTILE, NODES_PER_TILE), jnp.float32),
            pltpu.VMEM((NODES_PER_TILE,), jnp.float32),
            pltpu.VMEM_SHARED((NTILE, NPAD), jnp.float32),
            pltpu.VMEM_SHARED((NTILE, NPAD), jnp.float32),
            pltpu.SemaphoreType.DMA,
            pltpu.SemaphoreType.DMA,
        ],
    )


def kernel(h_prob, edge_index, edge_type, r_index, rel_emb, W):
    src = edge_index[0]
    dst = edge_index[1]
    ri32 = r_index.astype(jnp.int32)
    rb16 = jnp.broadcast_to(ri32[:, None], (B, 16))
    ri8 = jnp.concatenate(
        [jnp.broadcast_to(ri32[:, None], (B, D)),
         jnp.zeros((8 - B, D), jnp.int32)], axis=0)

    aux = pl.pallas_call(
        _prologue_body,
        out_shape=jax.ShapeDtypeStruct((8, D), jnp.float32),
    )(ri8, rel_emb, W)

    h_pad = jnp.concatenate(
        [h_prob, jnp.zeros((B, NPAD - N), jnp.float32)], axis=1)
    out = _build_sc_call()(h_pad, src, dst, edge_type, aux, rb16)
    return out[:, :N]


# speedup vs baseline: 1.0431x; 1.0431x over previous
import jax
import jax.numpy as jnp
from jax import lax
from jax.experimental import pallas as pl
from jax.experimental.pallas import tpu as pltpu
from jax.experimental.pallas import tpu_sc as plsc

N = 10000
E = 320000
D = 128
R = 64
B = 2

NPAD = 10240
NTILE = 16
EPT = E // NTILE
ECHUNK = 4000
NCHUNK = EPT // ECHUNK
CVECS = ECHUNK // 16
UNROLL = 10
NODES_PER_TILE = NPAD // NTILE

LN2 = 0.6931471805599453
MANT_MASK = 0x007FFFFF
ONE_BITS = 0x3F800000


def _prologue_body(ri_ref, rel_ref, w_ref, aux_ref):
    ri = ri_ref[...]
    rel = rel_ref[...]
    w = w_ref[...]
    iota_r = lax.broadcasted_iota(jnp.int32, (8, R), 1)
    oh = (ri[:, :R] == iota_r).astype(jnp.float32)
    query = jnp.dot(oh, rel, preferred_element_type=jnp.float32)
    wq = lax.dot_general(query, w, (((1,), (1,)), ((), ())),
                         preferred_element_type=jnp.float32)
    coef = lax.dot_general(query * wq, rel, (((1,), (1,)), ((), ())),
                           preferred_element_type=jnp.float32)
    q2 = jnp.sum(query * query, axis=1, keepdims=True)
    aux_ref[...] = jnp.concatenate(
        [coef, jnp.broadcast_to(q2, (8, 16)),
         jnp.zeros((8, D - R - 16), jnp.float32)], axis=1)


def _sc_body(h_hbm, src_hbm, dst_hbm, et_hbm, aux_hbm, rb_hbm,
             out_hbm,
             hv, cv, rvv,
             srcva, dstva, etva, srcvb, dstvb, etvb,
             sumacc, maxacc, done_s, bsum, bmax, outv,
             shsum, shmax, sema, semb):
    b = lax.axis_index("c")
    s = lax.axis_index("s")
    ebase = s * EPT

    bufsets = ((srcva, dstva, etva), (srcvb, dstvb, etvb))
    sems = (sema, semb)

    def start_chunk(ci):
        cbase = ebase + ci * ECHUNK
        bs = bufsets[ci % 2]
        sem = sems[ci % 2]
        return [pltpu.async_copy(src_hbm.at[pl.ds(cbase, ECHUNK)], bs[0], sem),
                pltpu.async_copy(dst_hbm.at[pl.ds(cbase, ECHUNK)], bs[1], sem),
                pltpu.async_copy(et_hbm.at[pl.ds(cbase, ECHUNK)], bs[2], sem)]

    handles = {0: start_chunk(0), 1: start_chunk(1)}

    with jax.named_scope("stage_tables"):
        pltpu.sync_copy(rb_hbm.at[b], rvv)
        pltpu.sync_copy(h_hbm.at[b], hv)
        pltpu.sync_copy(aux_hbm.at[b, pl.ds(0, R + 16)], cv)
    rv = rvv[...]

    zeros16 = jnp.zeros((16,), jnp.float32)
    neg16 = jnp.full((16,), -1e9, jnp.float32)

    def init_body(i, c):
        base = i * 128
        for u in range(8):
            sumacc[pl.ds(base + u * 16, 16)] = zeros16
            maxacc[pl.ds(base + u * 16, 16)] = neg16
        return c

    with jax.named_scope("init_accs"):
        lax.fori_loop(0, NPAD // 128, init_body, 0)

    def edge_vec(sv, dv, tv, off):
        s16 = sv[pl.ds(off, 16)]
        d16 = dv[pl.ds(off, 16)]
        t16 = tv[pl.ds(off, 16)]
        hb = plsc.load_gather(hv, [s16])
        cf = plsc.load_gather(cv, [t16])
        plsc.addupdate_scatter(sumacc, [d16], hb * cf)
        m = t16 == rv
        cur = plsc.load_gather(maxacc, [d16])
        plsc.store_scatter(maxacc, [d16], jnp.maximum(cur, hb), mask=m)
        chk = plsc.load_gather(maxacc, [d16])
        return m & (chk < hb)

    def fix_vec(sv, dv, tv, off):
        s16 = sv[pl.ds(off, 16)]
        d16 = dv[pl.ds(off, 16)]
        t16 = tv[pl.ds(off, 16)]
        hb = plsc.load_gather(hv, [s16])
        m = t16 == rv
        chk = plsc.load_gather(maxacc, [d16])
        pend = m & (chk < hb)
        done_s[...] = jnp.where(pend, 0, 1)

        def retry(k, c2):
            p = done_s[...] == 0

            @pl.when(jnp.any(p))
            def _():
                cur2 = plsc.load_gather(maxacc, [d16])
                plsc.store_scatter(maxacc, [d16],
                                   jnp.maximum(cur2, hb), mask=p)
                chk2 = plsc.load_gather(maxacc, [d16])
                done_s[...] = jnp.where(p & (chk2 < hb), 0, 1)

            return c2

        lax.fori_loop(0, 15, retry, 0)

    def make_group_body(bufset):
        sv, dv, tv = bufset

        def group_body(i, c):
            base = i * (16 * UNROLL)
            pends = [edge_vec(sv, dv, tv, base + u * 16)
                     for u in range(UNROLL)]
            anyp = pends[0]
            for p in pends[1:]:
                anyp = anyp | p

            @pl.when(jnp.any(anyp))
            def _():
                for u in range(UNROLL):
                    fix_vec(sv, dv, tv, base + u * 16)

            return c

        return group_body

    with jax.named_scope("edge_phase"):
        for ci in range(NCHUNK):
            for hdl in handles.pop(ci):
                hdl.wait()
            lax.fori_loop(0, CVECS // UNROLL,
                          make_group_body(bufsets[ci % 2]), 0)
            if ci + 2 < NCHUNK:
                handles[ci + 2] = start_chunk(ci + 2)

    with jax.named_scope("publish"):
        pltpu.sync_copy(sumacc, shsum.at[s])
        pltpu.sync_copy(maxacc, shmax.at[s])
        plsc.subcore_barrier()

    nbase = s * NODES_PER_TILE
    with jax.named_scope("combine_fetch"):
        pltpu.sync_copy(shsum.at[:, pl.ds(nbase, NODES_PER_TILE)], bsum)
        pltpu.sync_copy(shmax.at[:, pl.ds(nbase, NODES_PER_TILE)], bmax)

    q2 = cv[pl.ds(R, 16)]

    def comb_body(k, c):
        off = k * 16
        a = bsum[0, pl.ds(off, 16)]
        mx = bmax[0, pl.ds(off, 16)]
        for j in range(1, NTILE):
            a = a + bsum[j, pl.ds(off, 16)]
            mx = jnp.maximum(mx, bmax[j, pl.ds(off, 16)])

        hb16 = hv[pl.ds(nbase + off, 16)]
        score = a + hb16 * q2
        neural = 1.0 / (1.0 + jnp.exp(-score))
        sym = jnp.maximum(mx, 0.0)
        t = neural * sym
        u = (t + 1e-10) / (1.0 - t + 1e-10)
        bits = plsc.bitcast(u, jnp.int32)
        e = jnp.right_shift(bits, 23) - 127
        mant = plsc.bitcast(jnp.bitwise_or(jnp.bitwise_and(bits, MANT_MASK),
                                           ONE_BITS), jnp.float32)
        y = e.astype(jnp.float32) * LN2 + (mant - 1.0) * LN2
        y = y + u * jnp.exp(-y) - 1.0
        y = y + u * jnp.exp(-y) - 1.0
        y = y + u * jnp.exp(-y) - 1.0
        outv[pl.ds(off, 16)] = y
        return c

    with jax.named_scope("combine_epilogue"):
        lax.fori_loop(0, NODES_PER_TILE // 16, comb_body, 0)
        pltpu.sync_copy(outv, out_hbm.at[b, pl.ds(nbase, NODES_PER_TILE)])


def _build_sc_call():
    mesh = plsc.VectorSubcoreMesh(core_axis_name="c", subcore_axis_name="s")
    return pl.kernel(
        _sc_body,
        out_type=jax.ShapeDtypeStruct((B, NPAD), jnp.float32),
        mesh=mesh,
        compiler_params=pltpu.CompilerParams(needs_layout_passes=False),
        scratch_types=[
            pltpu.VMEM((NPAD,), jnp.float32),
            pltpu.VMEM((R + 16,), jnp.float32),
            pltpu.VMEM((16,), jnp.int32),
            pltpu.VMEM((ECHUNK,), jnp.int32),
            pltpu.VMEM((ECHUNK,), jnp.int32),
            pltpu.VMEM((ECHUNK,), jnp.int32),
            pltpu.VMEM((ECHUNK,), jnp.int32),
            pltpu.VMEM((ECHUNK,), jnp.int32),
            pltpu.VMEM((ECHUNK,), jnp.int32),
            pltpu.VMEM((NPAD,), jnp.float32),
            pltpu.VMEM((NPAD,), jnp.float32),
            pltpu.VMEM((16,), jnp.int32),
            pltpu.VMEM((NTILE, NODES_PER_TILE), jnp.float32),
            pltpu.VMEM((NTILE, NODES_PER_TILE), jnp.float32),
            pltpu.VMEM((NODES_PER_TILE,), jnp.float32),
            pltpu.VMEM_SHARED((NTILE, NPAD), jnp.float32),
            pltpu.VMEM_SHARED((NTILE, NPAD), jnp.float32),
            pltpu.SemaphoreType.DMA,
            pltpu.SemaphoreType.DMA,
        ],
    )


def kernel(h_prob, edge_index, edge_type, r_index, rel_emb, W):
    src = edge_index[0]
    dst = edge_index[1]
    ri32 = r_index.astype(jnp.int32)
    rb16 = jnp.broadcast_to(ri32[:, None], (B, 16))
    ri8 = jnp.concatenate(
        [jnp.broadcast_to(ri32[:, None], (B, D)),
         jnp.zeros((8 - B, D), jnp.int32)], axis=0)

    aux = pl.pallas_call(
        _prologue_body,
        out_shape=jax.ShapeDtypeStruct((8, D), jnp.float32),
    )(ri8, rel_emb, W)

    h_pad = jnp.concatenate(
        [h_prob, jnp.zeros((B, NPAD - N), jnp.float32)], axis=1)
    out = _build_sc_call()(h_pad, src, dst, edge_type, aux, rb16)
    return out[:, :N]


# speedup vs baseline: 1.0432x; 1.0001x over previous
import jax
import jax.numpy as jnp
from jax import lax
from jax.experimental import pallas as pl
from jax.experimental.pallas import tpu as pltpu
from jax.experimental.pallas import tpu_sc as plsc

N = 10000
E = 320000
D = 128
R = 64
B = 2

NPAD = 10240
NTILE = 16
EPT = E // NTILE
ECHUNK = 4000
NCHUNK = EPT // ECHUNK
CVECS = ECHUNK // 16
UNROLL = 10
NODES_PER_TILE = NPAD // NTILE

LN2 = 0.6931471805599453
MANT_MASK = 0x007FFFFF
ONE_BITS = 0x3F800000


def _prologue_body(ri_ref, rel_ref, w_ref, aux_ref):
    ri = ri_ref[...]
    rel = rel_ref[...]
    w = w_ref[...]
    iota_r = lax.broadcasted_iota(jnp.int32, (8, R), 1)
    oh = (ri[:, :R] == iota_r).astype(jnp.float32)
    query = jnp.dot(oh, rel, preferred_element_type=jnp.float32)
    wq = lax.dot_general(query, w, (((1,), (1,)), ((), ())),
                         preferred_element_type=jnp.float32)
    coef = lax.dot_general(query * wq, rel, (((1,), (1,)), ((), ())),
                           preferred_element_type=jnp.float32)
    q2 = jnp.sum(query * query, axis=1, keepdims=True)
    aux_ref[...] = jnp.concatenate(
        [coef, jnp.broadcast_to(q2, (8, 16)),
         jnp.zeros((8, D - R - 16), jnp.float32)], axis=1)


def _sc_body(h_hbm, src_hbm, dst_hbm, et_hbm, aux_hbm, rb_hbm,
             out_hbm,
             hv, cv, rvv,
             srcva, dstva, etva, srcvb, dstvb, etvb,
             sumacc, maxacc, maxaccb, done_s, bsum, bmax, outv,
             shsum, shmax, sema, semb, semt):
    b = lax.axis_index("c")
    s = lax.axis_index("s")
    ebase = s * EPT

    bufsets = ((srcva, dstva, etva), (srcvb, dstvb, etvb))
    sems = (sema, semb)

    def start_chunk(ci):
        cbase = ebase + ci * ECHUNK
        bs = bufsets[ci % 2]
        sem = sems[ci % 2]
        return [pltpu.async_copy(src_hbm.at[pl.ds(cbase, ECHUNK)], bs[0], sem),
                pltpu.async_copy(dst_hbm.at[pl.ds(cbase, ECHUNK)], bs[1], sem),
                pltpu.async_copy(et_hbm.at[pl.ds(cbase, ECHUNK)], bs[2], sem)]

    handles = {0: start_chunk(0), 1: start_chunk(1)}

    with jax.named_scope("stage_tables"):
        th = [pltpu.async_copy(rb_hbm.at[b], rvv, semt),
              pltpu.async_copy(h_hbm.at[b], hv, semt),
              pltpu.async_copy(aux_hbm.at[b, pl.ds(0, R + 16)], cv, semt)]

    zeros16 = jnp.zeros((16,), jnp.float32)
    neg16 = jnp.full((16,), -1e9, jnp.float32)

    def init_body(i, c):
        base = i * 128
        for u in range(8):
            sumacc[pl.ds(base + u * 16, 16)] = zeros16
            maxacc[pl.ds(base + u * 16, 16)] = neg16
            maxaccb[pl.ds(base + u * 16, 16)] = neg16
        return c

    with jax.named_scope("init_accs"):
        lax.fori_loop(0, NPAD // 128, init_body, 0)
        for hdl in th:
            hdl.wait()
    rv = rvv[...]

    def edge_vec(sv, dv, tv, mref, off):
        s16 = sv[pl.ds(off, 16)]
        d16 = dv[pl.ds(off, 16)]
        t16 = tv[pl.ds(off, 16)]
        hb = plsc.load_gather(hv, [s16])
        cf = plsc.load_gather(cv, [t16])
        plsc.addupdate_scatter(sumacc, [d16], hb * cf)
        m = t16 == rv
        cur = plsc.load_gather(mref, [d16])
        plsc.store_scatter(mref, [d16], jnp.maximum(cur, hb), mask=m)
        chk = plsc.load_gather(mref, [d16])
        return m & (chk < hb)

    def fix_vec(sv, dv, tv, mref, off):
        s16 = sv[pl.ds(off, 16)]
        d16 = dv[pl.ds(off, 16)]
        t16 = tv[pl.ds(off, 16)]
        hb = plsc.load_gather(hv, [s16])
        m = t16 == rv
        chk = plsc.load_gather(mref, [d16])
        pend = m & (chk < hb)
        done_s[...] = jnp.where(pend, 0, 1)

        def retry(k, c2):
            p = done_s[...] == 0

            @pl.when(jnp.any(p))
            def _():
                cur2 = plsc.load_gather(mref, [d16])
                plsc.store_scatter(mref, [d16],
                                   jnp.maximum(cur2, hb), mask=p)
                chk2 = plsc.load_gather(mref, [d16])
                done_s[...] = jnp.where(p & (chk2 < hb), 0, 1)

            return c2

        lax.fori_loop(0, 15, retry, 0)

    def make_group_body(bufset):
        sv, dv, tv = bufset

        def group_body(i, c):
            base = i * (16 * UNROLL)
            pends = [edge_vec(sv, dv, tv,
                              maxacc if u % 2 == 0 else maxaccb,
                              base + u * 16)
                     for u in range(UNROLL)]
            anyp = pends[0]
            for p in pends[1:]:
                anyp = anyp | p

            @pl.when(jnp.any(anyp))
            def _():
                for u in range(UNROLL):
                    fix_vec(sv, dv, tv,
                            maxacc if u % 2 == 0 else maxaccb,
                            base + u * 16)

            return c

        return group_body

    with jax.named_scope("edge_phase"):
        for ci in range(NCHUNK):
            for hdl in handles.pop(ci):
                hdl.wait()
            lax.fori_loop(0, CVECS // UNROLL,
                          make_group_body(bufsets[ci % 2]), 0)
            if ci + 2 < NCHUNK:
                handles[ci + 2] = start_chunk(ci + 2)

    def merge_body(i, c):
        base = i * 128
        for u in range(8):
            sl = pl.ds(base + u * 16, 16)
            maxacc[sl] = jnp.maximum(maxacc[sl], maxaccb[sl])
        return c

    with jax.named_scope("publish"):
        pltpu.sync_copy(sumacc, shsum.at[s])
        lax.fori_loop(0, NPAD // 128, merge_body, 0)
        pltpu.sync_copy(maxacc, shmax.at[s])
        plsc.subcore_barrier()

    nbase = s * NODES_PER_TILE
    with jax.named_scope("combine_fetch"):
        pltpu.sync_copy(shsum.at[:, pl.ds(nbase, NODES_PER_TILE)], bsum)
        pltpu.sync_copy(shmax.at[:, pl.ds(nbase, NODES_PER_TILE)], bmax)

    q2 = cv[pl.ds(R, 16)]

    def comb_body(k, c):
        for kk in range(2):
            off = k * 32 + kk * 16
            a = bsum[0, pl.ds(off, 16)]
            mx = bmax[0, pl.ds(off, 16)]
            for j in range(1, NTILE):
                a = a + bsum[j, pl.ds(off, 16)]
                mx = jnp.maximum(mx, bmax[j, pl.ds(off, 16)])

            hb16 = hv[pl.ds(nbase + off, 16)]
            score = a + hb16 * q2
            neural = 1.0 / (1.0 + jnp.exp(-score))
            sym = jnp.maximum(mx, 0.0)
            t = neural * sym
            u = (t + 1e-10) / (1.0 - t + 1e-10)
            bits = plsc.bitcast(u, jnp.int32)
            e = jnp.right_shift(bits, 23) - 127
            mant = plsc.bitcast(
                jnp.bitwise_or(jnp.bitwise_and(bits, MANT_MASK), ONE_BITS),
                jnp.float32)
            y = e.astype(jnp.float32) * LN2 + (mant - 1.0) * LN2
            y = y + u * jnp.exp(-y) - 1.0
            y = y + u * jnp.exp(-y) - 1.0
            y = y + u * jnp.exp(-y) - 1.0
            outv[pl.ds(off, 16)] = y
        return c

    with jax.named_scope("combine_epilogue"):
        lax.fori_loop(0, NODES_PER_TILE // 32, comb_body, 0)
        pltpu.sync_copy(outv, out_hbm.at[b, pl.ds(nbase, NODES_PER_TILE)])


def _build_sc_call():
    mesh = plsc.VectorSubcoreMesh(core_axis_name="c", subcore_axis_name="s")
    return pl.kernel(
        _sc_body,
        out_type=jax.ShapeDtypeStruct((B, NPAD), jnp.float32),
        mesh=mesh,
        compiler_params=pltpu.CompilerParams(needs_layout_passes=False),
        scratch_types=[
            pltpu.VMEM((NPAD,), jnp.float32),
            pltpu.VMEM((R + 16,), jnp.float32),
            pltpu.VMEM((16,), jnp.int32),
            pltpu.VMEM((ECHUNK,), jnp.int32),
            pltpu.VMEM((ECHUNK,), jnp.int32),
            pltpu.VMEM((ECHUNK,), jnp.int32),
            pltpu.VMEM((ECHUNK,), jnp.int32),
            pltpu.VMEM((ECHUNK,), jnp.int32),
            pltpu.VMEM((ECHUNK,), jnp.int32),
            pltpu.VMEM((NPAD,), jnp.float32),
            pltpu.VMEM((NPAD,), jnp.float32),
            pltpu.VMEM((NPAD,), jnp.float32),
            pltpu.VMEM((16,), jnp.int32),
            pltpu.VMEM((NTILE, NODES_PER_TILE), jnp.float32),
            pltpu.VMEM((NTILE, NODES_PER_TILE), jnp.float32),
            pltpu.VMEM((NODES_PER_TILE,), jnp.float32),
            pltpu.VMEM_SHARED((NTILE, NPAD), jnp.float32),
            pltpu.VMEM_SHARED((NTILE, NPAD), jnp.float32),
            pltpu.SemaphoreType.DMA,
            pltpu.SemaphoreType.DMA,
            pltpu.SemaphoreType.DMA,
        ],
    )


def kernel(h_prob, edge_index, edge_type, r_index, rel_emb, W):
    src = edge_index[0]
    dst = edge_index[1]
    ri32 = r_index.astype(jnp.int32)
    rb16 = jnp.broadcast_to(ri32[:, None], (B, 16))
    ri8 = jnp.concatenate(
        [jnp.broadcast_to(ri32[:, None], (B, D)),
         jnp.zeros((8 - B, D), jnp.int32)], axis=0)

    aux = pl.pallas_call(
        _prologue_body,
        out_shape=jax.ShapeDtypeStruct((8, D), jnp.float32),
    )(ri8, rel_emb, W)

    h_pad = jnp.concatenate(
        [h_prob, jnp.zeros((B, NPAD - N), jnp.float32)], axis=1)
    out = _build_sc_call()(h_pad, src, dst, edge_type, aux, rb16)
    return out[:, :N]


# speedup vs baseline: 1.0443x; 1.0011x over previous
import jax
import jax.numpy as jnp
from jax import lax
from jax.experimental import pallas as pl
from jax.experimental.pallas import tpu as pltpu
from jax.experimental.pallas import tpu_sc as plsc

N = 10000
E = 320000
D = 128
R = 64
B = 2

NPAD = 10240
NTILE = 16
EPT = E // NTILE
ECHUNK = 4000
NCHUNK = EPT // ECHUNK
CVECS = ECHUNK // 16
UNROLL = 10
NODES_PER_TILE = NPAD // NTILE

LN2 = 0.6931471805599453
MANT_MASK = 0x007FFFFF
ONE_BITS = 0x3F800000


def _prologue_body(ri_ref, rel_ref, w_ref, aux_ref):
    ri = ri_ref[...]
    rel = rel_ref[...]
    w = w_ref[...]
    iota_r = lax.broadcasted_iota(jnp.int32, (8, R), 1)
    oh = (ri[:, :R] == iota_r).astype(jnp.float32)
    query = jnp.dot(oh, rel, preferred_element_type=jnp.float32)
    wq = lax.dot_general(query, w, (((1,), (1,)), ((), ())),
                         preferred_element_type=jnp.float32)
    coef = lax.dot_general(query * wq, rel, (((1,), (1,)), ((), ())),
                           preferred_element_type=jnp.float32)
    q2 = jnp.sum(query * query, axis=1, keepdims=True)
    aux_ref[...] = jnp.concatenate(
        [coef, jnp.broadcast_to(q2, (8, 16)),
         jnp.zeros((8, D - R - 16), jnp.float32)], axis=1)


def _sc_body(h_hbm, src_hbm, dst_hbm, et_hbm, aux_hbm, rb_hbm,
             out_hbm,
             hv, cv, rvv,
             srcva, dstva, etva, srcvb, dstvb, etvb,
             sumacc, maxacc, maxaccb, done_s, bsum, bmax, outv,
             shsum, shmax, sema, semb, semt):
    b = lax.axis_index("c")
    s = lax.axis_index("s")
    ebase = s * EPT

    bufsets = ((srcva, dstva, etva), (srcvb, dstvb, etvb))
    sems = (sema, semb)

    def start_chunk(ci):
        cbase = ebase + ci * ECHUNK
        bs = bufsets[ci % 2]
        sem = sems[ci % 2]
        return [pltpu.async_copy(src_hbm.at[pl.ds(cbase, ECHUNK)], bs[0], sem),
                pltpu.async_copy(dst_hbm.at[pl.ds(cbase, ECHUNK)], bs[1], sem),
                pltpu.async_copy(et_hbm.at[pl.ds(cbase, ECHUNK)], bs[2], sem)]

    handles = {0: start_chunk(0), 1: start_chunk(1)}

    th = [pltpu.async_copy(rb_hbm.at[b], rvv, semt),
          pltpu.async_copy(h_hbm.at[b], hv, semt),
          pltpu.async_copy(aux_hbm.at[b, pl.ds(0, R + 16)], cv, semt)]

    zeros16 = jnp.zeros((16,), jnp.float32)
    neg16 = jnp.full((16,), -1e9, jnp.float32)

    def init_body(i, c):
        base = i * 128
        for u in range(8):
            sumacc[pl.ds(base + u * 16, 16)] = zeros16
            maxacc[pl.ds(base + u * 16, 16)] = neg16
            maxaccb[pl.ds(base + u * 16, 16)] = neg16
        return c

    lax.fori_loop(0, NPAD // 128, init_body, 0)
    for hdl in th:
        hdl.wait()
    rv = rvv[...]

    def edge_vec(sv, dv, tv, mref, off):
        s16 = sv[pl.ds(off, 16)]
        d16 = dv[pl.ds(off, 16)]
        t16 = tv[pl.ds(off, 16)]
        hb = plsc.load_gather(hv, [s16])
        cf = plsc.load_gather(cv, [t16])
        plsc.addupdate_scatter(sumacc, [d16], hb * cf)
        m = t16 == rv
        cur = plsc.load_gather(mref, [d16])
        plsc.store_scatter(mref, [d16], jnp.maximum(cur, hb), mask=m)
        chk = plsc.load_gather(mref, [d16])
        return m & (chk < hb)

    def fix_vec(sv, dv, tv, mref, off):
        s16 = sv[pl.ds(off, 16)]
        d16 = dv[pl.ds(off, 16)]
        t16 = tv[pl.ds(off, 16)]
        hb = plsc.load_gather(hv, [s16])
        m = t16 == rv
        chk = plsc.load_gather(mref, [d16])
        pend = m & (chk < hb)
        done_s[...] = jnp.where(pend, 0, 1)

        def retry(k, c2):
            p = done_s[...] == 0

            @pl.when(jnp.any(p))
            def _():
                cur2 = plsc.load_gather(mref, [d16])
                plsc.store_scatter(mref, [d16],
                                   jnp.maximum(cur2, hb), mask=p)
                chk2 = plsc.load_gather(mref, [d16])
                done_s[...] = jnp.where(p & (chk2 < hb), 0, 1)

            return c2

        lax.fori_loop(0, 15, retry, 0)

    def make_group_body(bufset):
        sv, dv, tv = bufset

        def group_body(i, c):
            base = i * (16 * UNROLL)
            pends = [edge_vec(sv, dv, tv,
                              maxacc if u % 2 == 0 else maxaccb,
                              base + u * 16)
                     for u in range(UNROLL)]
            anyp = pends[0]
            for p in pends[1:]:
                anyp = anyp | p

            @pl.when(jnp.any(anyp))
            def _():
                for u in range(UNROLL):
                    fix_vec(sv, dv, tv,
                            maxacc if u % 2 == 0 else maxaccb,
                            base + u * 16)

            return c

        return group_body

    for ci in range(NCHUNK):
        for hdl in handles.pop(ci):
            hdl.wait()
        lax.fori_loop(0, CVECS // UNROLL,
                      make_group_body(bufsets[ci % 2]), 0)
        if ci + 2 < NCHUNK:
            handles[ci + 2] = start_chunk(ci + 2)

    def merge_body(i, c):
        base = i * 128
        for u in range(8):
            sl = pl.ds(base + u * 16, 16)
            maxacc[sl] = jnp.maximum(maxacc[sl], maxaccb[sl])
        return c

    pltpu.sync_copy(sumacc, shsum.at[s])
    lax.fori_loop(0, NPAD // 128, merge_body, 0)
    pltpu.sync_copy(maxacc, shmax.at[s])
    plsc.subcore_barrier()

    nbase = s * NODES_PER_TILE
    pltpu.sync_copy(shsum.at[:, pl.ds(nbase, NODES_PER_TILE)], bsum)
    pltpu.sync_copy(shmax.at[:, pl.ds(nbase, NODES_PER_TILE)], bmax)

    q2 = cv[pl.ds(R, 16)]

    def comb_body(k, c):
        for kk in range(2):
            off = k * 32 + kk * 16
            a = bsum[0, pl.ds(off, 16)]
            mx = bmax[0, pl.ds(off, 16)]
            for j in range(1, NTILE):
                a = a + bsum[j, pl.ds(off, 16)]
                mx = jnp.maximum(mx, bmax[j, pl.ds(off, 16)])

            hb16 = hv[pl.ds(nbase + off, 16)]
            score = a + hb16 * q2
            neural = 1.0 / (1.0 + jnp.exp(-score))
            sym = jnp.maximum(mx, 0.0)
            t = neural * sym
            u = (t + 1e-10) / (1.0 - t + 1e-10)
            bits = plsc.bitcast(u, jnp.int32)
            e = jnp.right_shift(bits, 23) - 127
            mant = plsc.bitcast(
                jnp.bitwise_or(jnp.bitwise_and(bits, MANT_MASK), ONE_BITS),
                jnp.float32)
            y = e.astype(jnp.float32) * LN2 + (mant - 1.0) * LN2
            y = y + u * jnp.exp(-y) - 1.0
            y = y + u * jnp.exp(-y) - 1.0
            y = y + u * jnp.exp(-y) - 1.0
            outv[pl.ds(off, 16)] = y
        return c

    lax.fori_loop(0, NODES_PER_TILE // 32, comb_body, 0)
    pltpu.sync_copy(outv, out_hbm.at[b, pl.ds(nbase, NODES_PER_TILE)])


def _build_sc_call():
    mesh = plsc.VectorSubcoreMesh(core_axis_name="c", subcore_axis_name="s")
    return pl.kernel(
        _sc_body,
        out_type=jax.ShapeDtypeStruct((B, NPAD), jnp.float32),
        mesh=mesh,
        compiler_params=pltpu.CompilerParams(needs_layout_passes=False),
        scratch_types=[
            pltpu.VMEM((NPAD,), jnp.float32),
            pltpu.VMEM((R + 16,), jnp.float32),
            pltpu.VMEM((16,), jnp.int32),
            pltpu.VMEM((ECHUNK,), jnp.int32),
            pltpu.VMEM((ECHUNK,), jnp.int32),
            pltpu.VMEM((ECHUNK,), jnp.int32),
            pltpu.VMEM((ECHUNK,), jnp.int32),
            pltpu.VMEM((ECHUNK,), jnp.int32),
            pltpu.VMEM((ECHUNK,), jnp.int32),
            pltpu.VMEM((NPAD,), jnp.float32),
            pltpu.VMEM((NPAD,), jnp.float32),
            pltpu.VMEM((NPAD,), jnp.float32),
            pltpu.VMEM((16,), jnp.int32),
            pltpu.VMEM((NTILE, NODES_PER_TILE), jnp.float32),
            pltpu.VMEM((NTILE, NODES_PER_TILE), jnp.float32),
            pltpu.VMEM((NODES_PER_TILE,), jnp.float32),
            pltpu.VMEM_SHARED((NTILE, NPAD), jnp.float32),
            pltpu.VMEM_SHARED((NTILE, NPAD), jnp.float32),
            pltpu.SemaphoreType.DMA,
            pltpu.SemaphoreType.DMA,
            pltpu.SemaphoreType.DMA,
        ],
    )


def kernel(h_prob, edge_index, edge_type, r_index, rel_emb, W):
    src = edge_index[0]
    dst = edge_index[1]
    ri32 = r_index.astype(jnp.int32)
    rb16 = jnp.broadcast_to(ri32[:, None], (B, 16))
    ri8 = jnp.concatenate(
        [jnp.broadcast_to(ri32[:, None], (B, D)),
         jnp.zeros((8 - B, D), jnp.int32)], axis=0)

    aux = pl.pallas_call(
        _prologue_body,
        out_shape=jax.ShapeDtypeStruct((8, D), jnp.float32),
    )(ri8, rel_emb, W)

    h_pad = jnp.concatenate(
        [h_prob, jnp.zeros((B, NPAD - N), jnp.float32)], axis=1)
    out = _build_sc_call()(h_pad, src, dst, edge_type, aux, rb16)
    return out[:, :N]
